# baseline (speedup 1.0000x reference)
import functools

import jax
import jax.numpy as jnp
import numpy as np
from jax.experimental import pallas as pl
from jax.experimental.pallas import tpu as pltpu

_N = 10000
_K = 16

_BIG_F = 3.0e38
_BIG_I = 2**30


def _knn_kernel_body(nc, xr_ref, xc_ref, idx_ref, rv_ref, ri_ref, cv_ref, *,
                     bm, bn, n_valid):
    j = pl.program_id(1)
    i = pl.program_id(0)

    @pl.when(j == 0)
    def _init():
        rv_ref[...] = jnp.full((bm, _K), _BIG_F, jnp.float32)
        ri_ref[...] = jnp.zeros((bm, _K), jnp.int32)

    xr = xr_ref[...]
    xc = xc_ref[...]
    d2c = jnp.sum(xc * xc, axis=1)
    dots = jax.lax.dot_general(
        xr, xc, (((1,), (1,)), ((), ())), preferred_element_type=jnp.float32
    )
    dist = d2c[None, :] - 2.0 * dots
    col_ids = j * bn + jax.lax.broadcasted_iota(jnp.int32, (bm, bn), 1)
    row_ids = i * bm + jax.lax.broadcasted_iota(jnp.int32, (bm, bn), 0)
    invalid = (col_ids == row_ids) | (col_ids >= n_valid)
    cv_ref[:, :_K] = rv_ref[...]
    cv_ref[:, _K:] = jnp.where(invalid, _BIG_F, dist)
    cand_i = jnp.concatenate([ri_ref[...], col_ids], axis=1)
    lane = jax.lax.broadcasted_iota(jnp.int32, (bm, _K), 1)

    def _extract(t, carry):
        nv, ni = carry
        cv = cv_ref[...]
        m = jnp.min(cv, axis=1, keepdims=True)
        eq = cv == m
        sel = jnp.min(jnp.where(eq, cand_i, _BIG_I), axis=1, keepdims=True)
        cv_ref[...] = jnp.where(eq & (cand_i == sel), _BIG_F, cv)
        nv = jnp.where(lane == t, m, nv)
        ni = jnp.where(lane == t, sel, ni)
        return nv, ni

    nv0 = jnp.full((bm, _K), _BIG_F, jnp.float32)
    ni0 = jnp.zeros((bm, _K), jnp.int32)
    nv, ni = jax.lax.fori_loop(0, _K, _extract, (nv0, ni0))
    rv_ref[...] = nv
    ri_ref[...] = ni

    @pl.when(j == nc - 1)
    def _write():
        idx_ref[...] = ni


@functools.partial(jax.jit, static_argnames=("interpret",))
def _knn_idx_pallas(x, interpret=False):
    n, d = x.shape
    bm = 128
    bn = 256
    npad = ((n + bn - 1) // bn) * bn
    xp = jnp.pad(x, ((0, npad - n), (0, 0)))
    nr = npad // bm
    nc = npad // bn
    body = functools.partial(_knn_kernel_body, nc, bm=bm, bn=bn, n_valid=n)
    idx = pl.pallas_call(
        body,
        grid=(nr, nc),
        in_specs=[
            pl.BlockSpec((bm, d), lambda i, j: (i, 0)),
            pl.BlockSpec((bn, d), lambda i, j: (j, 0)),
        ],
        out_specs=pl.BlockSpec((bm, _K), lambda i, j: (i, 0)),
        out_shape=jax.ShapeDtypeStruct((npad, _K), jnp.int32),
        scratch_shapes=[
            pltpu.VMEM((bm, _K), jnp.float32),
            pltpu.VMEM((bm, _K), jnp.int32),
            pltpu.VMEM((bm, bn + _K), jnp.float32),
        ],
        interpret=interpret,
    )(xp, xp)
    return idx[:n]


def _gatv2_layer(x, Wl, Wr, a, b, interpret=False):
    idx = _knn_idx_pallas(x, interpret=interpret)
    hl = x @ Wl
    hr = x @ Wr
    hr_n = jnp.take(hr, idx, axis=0)
    m = jax.nn.leaky_relu(hl[:, None, :] + hr_n, negative_slope=0.2)
    e = jnp.einsum("nkd,d->nk", m, a)
    alpha = jax.nn.softmax(e, axis=1)
    return jnp.sum(alpha[:, :, None] * hr_n, axis=1) + b


def kernel(x, Wl1, Wr1, a1, b1, Wl2, Wr2, a2, b2, Wl3, Wr3, a3, b3,
           Wl4, Wr4, a4, b4, Wm1, bm1, Wm2, bm2, Wm3, bm3):
    out_list = [x]
    params = [(Wl1, Wr1, a1, b1), (Wl2, Wr2, a2, b2), (Wl3, Wr3, a3, b3),
              (Wl4, Wr4, a4, b4)]
    for (Wl, Wr, a, b) in params:
        out_list.append(_gatv2_layer(out_list[-1], Wl, Wr, a, b))
    h = jnp.concatenate(out_list, axis=1)
    h = jax.nn.relu(h @ Wm1 + bm1)
    h = jax.nn.relu(h @ Wm2 + bm2)
    h = h @ Wm3 + bm3
    return h


# speedup vs baseline: 10.9889x; 10.9889x over previous
import functools

import jax
import jax.numpy as jnp
import numpy as np
from jax.experimental import pallas as pl
from jax.experimental.pallas import tpu as pltpu

_N = 10000
_K = 16

_BIG_F = 3.0e38
_BIG_I = 2**30


def _knn_kernel_body(nc, xr_ref, xc_ref, idx_ref, rv_ref, ri_ref, cv_ref,
                     ci_ref, *, bm, bn, n_valid):
    j = pl.program_id(1)
    i = pl.program_id(0)

    @pl.when(j == 0)
    def _init():
        rv_ref[...] = jnp.full((_K, bm), _BIG_F, jnp.float32)
        ri_ref[...] = jnp.zeros((_K, bm), jnp.int32)

    xr = xr_ref[...]
    xc = xc_ref[...]
    d2c = jnp.sum(xc * xc, axis=1, keepdims=True)
    dots = jax.lax.dot_general(
        xc, xr, (((1,), (1,)), ((), ())), preferred_element_type=jnp.float32
    )
    dist = d2c - 2.0 * dots
    col_ids = j * bn + jax.lax.broadcasted_iota(jnp.int32, (bn, bm), 0)
    row_ids = i * bm + jax.lax.broadcasted_iota(jnp.int32, (bn, bm), 1)
    invalid = (col_ids == row_ids) | (col_ids >= n_valid)
    cv_ref[:_K, :] = rv_ref[...]
    cv_ref[_K:, :] = jnp.where(invalid, _BIG_F, dist)
    ci_ref[:_K, :] = ri_ref[...]
    ci_ref[_K:, :] = col_ids

    ns = bn + _K
    chunks = []
    s = 0
    while s < ns:
        chunks.append((s, min(64, ns - s)))
        s += min(64, ns - s)
    siota = jax.lax.broadcasted_iota(jnp.int32, (_K, bm), 0)

    def _extract(t, carry):
        nv, ni = carry
        m = jnp.full((1, bm), _BIG_F, jnp.float32)
        for (s, sz) in chunks:
            m = jnp.minimum(
                m, jnp.min(cv_ref[pl.ds(s, sz), :], axis=0, keepdims=True))
        sel = jnp.full((1, bm), _BIG_I, jnp.int32)
        for (s, sz) in chunks:
            cvc = cv_ref[pl.ds(s, sz), :]
            cic = ci_ref[pl.ds(s, sz), :]
            sel = jnp.minimum(
                sel,
                jnp.min(jnp.where(cvc == m, cic, _BIG_I), axis=0,
                        keepdims=True))
        for (s, sz) in chunks:
            cvc = cv_ref[pl.ds(s, sz), :]
            cic = ci_ref[pl.ds(s, sz), :]
            cv_ref[pl.ds(s, sz), :] = jnp.where(
                (cvc == m) & (cic == sel), _BIG_F, cvc)
        nv = jnp.where(siota == t, m, nv)
        ni = jnp.where(siota == t, sel, ni)
        return nv, ni

    nv0 = jnp.full((_K, bm), _BIG_F, jnp.float32)
    ni0 = jnp.zeros((_K, bm), jnp.int32)
    nv, ni = jax.lax.fori_loop(0, _K, _extract, (nv0, ni0))
    rv_ref[...] = nv
    ri_ref[...] = ni

    @pl.when(j == nc - 1)
    def _write():
        idx_ref[...] = ni


@functools.partial(jax.jit, static_argnames=("interpret",))
def _knn_idx_pallas(x, interpret=False):
    n, d = x.shape
    bm = 128
    bn = 512
    npad = ((n + bn - 1) // bn) * bn
    xp = jnp.pad(x, ((0, npad - n), (0, 0)))
    nr = npad // bm
    nc = npad // bn
    body = functools.partial(_knn_kernel_body, nc, bm=bm, bn=bn, n_valid=n)
    idx = pl.pallas_call(
        body,
        grid=(nr, nc),
        in_specs=[
            pl.BlockSpec((bm, d), lambda i, j: (i, 0)),
            pl.BlockSpec((bn, d), lambda i, j: (j, 0)),
        ],
        out_specs=pl.BlockSpec((_K, bm), lambda i, j: (0, i)),
        out_shape=jax.ShapeDtypeStruct((_K, npad), jnp.int32),
        scratch_shapes=[
            pltpu.VMEM((_K, bm), jnp.float32),
            pltpu.VMEM((_K, bm), jnp.int32),
            pltpu.VMEM((bn + _K, bm), jnp.float32),
            pltpu.VMEM((bn + _K, bm), jnp.int32),
        ],
        interpret=interpret,
    )(xp, xp)
    return idx[:, :n].T


def _gatv2_layer(x, Wl, Wr, a, b, interpret=False):
    idx = _knn_idx_pallas(x, interpret=interpret)
    hl = x @ Wl
    hr = x @ Wr
    hr_n = jnp.take(hr, idx, axis=0)
    m = jax.nn.leaky_relu(hl[:, None, :] + hr_n, negative_slope=0.2)
    e = jnp.einsum("nkd,d->nk", m, a)
    alpha = jax.nn.softmax(e, axis=1)
    return jnp.sum(alpha[:, :, None] * hr_n, axis=1) + b


def kernel(x, Wl1, Wr1, a1, b1, Wl2, Wr2, a2, b2, Wl3, Wr3, a3, b3,
           Wl4, Wr4, a4, b4, Wm1, bm1, Wm2, bm2, Wm3, bm3):
    out_list = [x]
    params = [(Wl1, Wr1, a1, b1), (Wl2, Wr2, a2, b2), (Wl3, Wr3, a3, b3),
              (Wl4, Wr4, a4, b4)]
    for (Wl, Wr, a, b) in params:
        out_list.append(_gatv2_layer(out_list[-1], Wl, Wr, a, b))
    h = jnp.concatenate(out_list, axis=1)
    h = jax.nn.relu(h @ Wm1 + bm1)
    h = jax.nn.relu(h @ Wm2 + bm2)
    h = h @ Wm3 + bm3
    return h
